# baseline (speedup 1.0000x reference)
import jax
import jax.numpy as jnp
from jax.experimental import pallas as pl
from jax.experimental.pallas import tpu as pltpu

BS = 32
C = 64
T = 256
ED = 32
N = BS * C
UNROLL = 8


def _dot_t(a, b):
    return jax.lax.dot_general(a, b, (((1,), (1,)), ((), ())),
                               preferred_element_type=jnp.float32)


def _moe_body(xl_ref, xr_ref, wp_ref, bp_ref, cen_ref,
              we_ref, be_ref,
              lls_ref, llb_ref, lrs_ref, lrb_ref,
              ol_ref, or_ref,
              g_ref, xlb_ref, xrb_ref, accl_ref, accr_ref):
    e = pl.program_id(0)
    ids = jax.lax.broadcasted_iota(jnp.int32, (N, C), 1)

    @pl.when(e == 0)
    def _router():
        xl = xl_ref[...]
        xr = xr_ref[...]
        wp = wp_ref[...]
        xp = (_dot_t(xl, wp[:, :T]) + _dot_t(xr, wp[:, T:])
              + bp_ref[...])
        n = jnp.sqrt(jnp.sum(xp * xp, axis=-1, keepdims=True))
        xp = xp / jnp.maximum(n, 1e-12)
        cen = cen_ref[...]
        cn = jnp.sqrt(jnp.sum(cen * cen, axis=-1, keepdims=True))
        cen = cen / jnp.maximum(cn, 1e-12)
        sims = _dot_t(xp, cen)
        v1 = jnp.max(sims, axis=-1, keepdims=True)
        i1 = jnp.min(jnp.where(sims == v1, ids, C), axis=-1, keepdims=True)
        sims2 = jnp.where(ids == i1, -jnp.inf, sims)
        v2 = jnp.max(sims2, axis=-1, keepdims=True)
        i2 = jnp.min(jnp.where(sims2 == v2, ids, C), axis=-1, keepdims=True)
        g_ref[...] = (jnp.where(ids == i1, 1.0, 0.0)
                      + jnp.where(ids == i2, 1.0, 0.0)).astype(jnp.bfloat16)
        xlb_ref[...] = xl.astype(jnp.bfloat16)
        xrb_ref[...] = xr.astype(jnp.bfloat16)
        accl_ref[...] = jnp.zeros_like(accl_ref)
        accr_ref[...] = jnp.zeros_like(accr_ref)

    dl = []
    dr = []
    for u in range(UNROLL):
        w = we_ref[u]
        eu = e * UNROLL + u
        g = jnp.sum(jnp.where(ids == eu, g_ref[...], jnp.bfloat16(0)),
                    axis=-1, keepdims=True)
        dl.append(_dot_t(g * xlb_ref[...], w))
        dr.append(_dot_t(g * xrb_ref[...], w))
    accl_ref[...] += sum(dl[1:], dl[0])
    accr_ref[...] += sum(dr[1:], dr[0])

    @pl.when(e == C // UNROLL - 1)
    def _finish():
        bias = jax.lax.dot_general(g_ref[...].astype(jnp.float32), be_ref[...],
                                   (((1,), (0,)), ((), ())),
                                   preferred_element_type=jnp.float32)
        for acc_ref, x_ref, s_ref, b_ref, o_ref in (
                (accl_ref, xl_ref, lls_ref, llb_ref, ol_ref),
                (accr_ref, xr_ref, lrs_ref, lrb_ref, or_ref)):
            a = acc_ref[...] + bias
            mu = jnp.mean(a, axis=-1, keepdims=True)
            d = a - mu
            var = jnp.mean(d * d, axis=-1, keepdims=True)
            yn = d * jax.lax.rsqrt(var + 1e-5)
            o_ref[...] = yn * s_ref[...] + b_ref[...] + x_ref[...]


def kernel(x_l, x_r, W_proj, b_proj, expert_centers, W_experts, b_experts,
           ln_l_scale, ln_l_bias, ln_r_scale, ln_r_bias):
    xl = x_l.reshape(N, T)
    xr = x_r.reshape(N, T)
    full = lambda shape: pl.BlockSpec(shape, lambda e: (0,) * len(shape))
    out_l, out_r = pl.pallas_call(
        _moe_body,
        grid=(C // UNROLL,),
        in_specs=[
            full((N, T)), full((N, T)),
            full((ED, 2 * T)), full((1, ED)), full((C, ED)),
            pl.BlockSpec((UNROLL, T, T), lambda e: (e, 0, 0)),
            full((C, T)),
            full((1, T)), full((1, T)), full((1, T)), full((1, T)),
        ],
        out_specs=[full((N, T)), full((N, T))],
        out_shape=[jax.ShapeDtypeStruct((N, T), jnp.float32),
                   jax.ShapeDtypeStruct((N, T), jnp.float32)],
        scratch_shapes=[
            pltpu.VMEM((N, C), jnp.bfloat16),
            pltpu.VMEM((N, T), jnp.bfloat16),
            pltpu.VMEM((N, T), jnp.bfloat16),
            pltpu.VMEM((N, T), jnp.float32),
            pltpu.VMEM((N, T), jnp.float32),
        ],
    )(xl, xr,
      W_proj, b_proj.reshape(1, ED), expert_centers,
      W_experts.astype(jnp.bfloat16), b_experts,
      ln_l_scale.reshape(1, T), ln_l_bias.reshape(1, T),
      ln_r_scale.reshape(1, T), ln_r_bias.reshape(1, T))
    return (out_l.reshape(BS, C, T), out_r.reshape(BS, C, T))


# speedup vs baseline: 1.1967x; 1.1967x over previous
import jax
import jax.numpy as jnp
from jax.experimental import pallas as pl
from jax.experimental.pallas import tpu as pltpu

BS = 32
C = 64
T = 256
ED = 32
N = BS * C
UNROLL = 16


def _dot_t(a, b):
    return jax.lax.dot_general(a, b, (((1,), (1,)), ((), ())),
                               preferred_element_type=jnp.float32)


def _moe_body(xl_ref, xr_ref, wp_ref, bp_ref, cen_ref,
              we_ref, be_ref,
              lls_ref, llb_ref, lrs_ref, lrb_ref,
              ol_ref, or_ref,
              g_ref, accl_ref, accr_ref):
    e = pl.program_id(0)
    ids = jax.lax.broadcasted_iota(jnp.int32, (N, C), 1)

    @pl.when(e == 0)
    def _router():
        xl = xl_ref[...]
        xr = xr_ref[...]
        wp = wp_ref[...]
        xp = (_dot_t(xl, wp[:, :T]) + _dot_t(xr, wp[:, T:])
              + bp_ref[...])
        n = jnp.sqrt(jnp.sum(xp * xp, axis=-1, keepdims=True))
        xp = xp / jnp.maximum(n, 1e-12)
        cen = cen_ref[...]
        cn = jnp.sqrt(jnp.sum(cen * cen, axis=-1, keepdims=True))
        cen = cen / jnp.maximum(cn, 1e-12)
        sims = _dot_t(xp, cen)
        v1 = jnp.max(sims, axis=-1, keepdims=True)
        i1 = jnp.min(jnp.where(sims == v1, ids, C), axis=-1, keepdims=True)
        sims2 = jnp.where(ids == i1, -jnp.inf, sims)
        v2 = jnp.max(sims2, axis=-1, keepdims=True)
        i2 = jnp.min(jnp.where(sims2 == v2, ids, C), axis=-1, keepdims=True)
        g_ref[...] = (jnp.where(ids == i1, 1.0, 0.0)
                      + jnp.where(ids == i2, 1.0, 0.0))
        accl_ref[...] = jnp.zeros_like(accl_ref)
        accr_ref[...] = jnp.zeros_like(accr_ref)

    dl = []
    dr = []
    for u in range(UNROLL):
        w = we_ref[u]
        eu = e * UNROLL + u
        g = jnp.sum(jnp.where(ids == eu, g_ref[...], 0.0),
                    axis=-1, keepdims=True)
        dl.append(_dot_t(g * xl_ref[...], w))
        dr.append(_dot_t(g * xr_ref[...], w))
    accl_ref[...] += sum(dl[1:], dl[0])
    accr_ref[...] += sum(dr[1:], dr[0])

    @pl.when(e == C // UNROLL - 1)
    def _finish():
        bias = jax.lax.dot_general(g_ref[...], be_ref[...],
                                   (((1,), (0,)), ((), ())),
                                   preferred_element_type=jnp.float32)
        for acc_ref, x_ref, s_ref, b_ref, o_ref in (
                (accl_ref, xl_ref, lls_ref, llb_ref, ol_ref),
                (accr_ref, xr_ref, lrs_ref, lrb_ref, or_ref)):
            a = acc_ref[...] + bias
            mu = jnp.mean(a, axis=-1, keepdims=True)
            d = a - mu
            var = jnp.mean(d * d, axis=-1, keepdims=True)
            yn = d * jax.lax.rsqrt(var + 1e-5)
            o_ref[...] = yn * s_ref[...] + b_ref[...] + x_ref[...]


def kernel(x_l, x_r, W_proj, b_proj, expert_centers, W_experts, b_experts,
           ln_l_scale, ln_l_bias, ln_r_scale, ln_r_bias):
    xl = x_l.reshape(N, T)
    xr = x_r.reshape(N, T)
    full = lambda shape: pl.BlockSpec(shape, lambda e: (0,) * len(shape))
    out_l, out_r = pl.pallas_call(
        _moe_body,
        grid=(C // UNROLL,),
        in_specs=[
            full((N, T)), full((N, T)),
            full((ED, 2 * T)), full((1, ED)), full((C, ED)),
            pl.BlockSpec((UNROLL, T, T), lambda e: (e, 0, 0)),
            full((C, T)),
            full((1, T)), full((1, T)), full((1, T)), full((1, T)),
        ],
        out_specs=[full((N, T)), full((N, T))],
        out_shape=[jax.ShapeDtypeStruct((N, T), jnp.float32),
                   jax.ShapeDtypeStruct((N, T), jnp.float32)],
        scratch_shapes=[
            pltpu.VMEM((N, C), jnp.float32),
            pltpu.VMEM((N, T), jnp.float32),
            pltpu.VMEM((N, T), jnp.float32),
        ],
    )(xl, xr,
      W_proj, b_proj.reshape(1, ED), expert_centers,
      W_experts, b_experts,
      ln_l_scale.reshape(1, T), ln_l_bias.reshape(1, T),
      ln_r_scale.reshape(1, T), ln_r_bias.reshape(1, T))
    return (out_l.reshape(BS, C, T), out_r.reshape(BS, C, T))
